# baseline (speedup 1.0000x reference)
import jax
import jax.numpy as jnp
from jax import lax
from jax.experimental import pallas as pl
from jax.experimental.pallas import tpu as pltpu
from jax.experimental.pallas import tpu_sc as plsc

B = 16384
D = 64
E = 1024
N_USER = 1000000
N_ITEM = 1000000
NC = 2
NS = 16
NW = NC * NS
BPW = B // NW
FH = D // 2

_mesh = plsc.VectorSubcoreMesh(core_axis_name="c", subcore_axis_name="s")


def _wid():
    return lax.axis_index("s") * NC + lax.axis_index("c")


def _sc_gather3_body(uid_h, pos_h, neg_h, userf_h, itemf_h,
                     ue_o, po_o, no_o,
                     idxv, fidx, vals, semg, semw):
    wid = _wid()
    base = wid * BPW
    for t, ih in enumerate((uid_h, pos_h, neg_h)):
        pltpu.sync_copy(ih.at[pl.ds(base, BPW)], idxv.at[t])
    jobs = ((0, userf_h, ue_o, N_USER), (1, itemf_h, po_o, N_ITEM + 1),
            (2, itemf_h, no_o, N_ITEM + 1))
    prev_writes = []
    for t, flat, out, stride in jobs:
        for half in range(2):
            c0 = half * FH

            def body(k, carry, t=t, c0=c0, stride=stride):
                v = idxv[t, pl.ds(k * 16, 16)]
                for c in range(FH):
                    fidx[pl.ds(c * BPW + k * 16, 16)] = v + (c0 + c) * stride
                return carry

            lax.fori_loop(0, BPW // 16, body, 0)
            for w in prev_writes:
                w.wait()
            pltpu.async_copy(flat.at[fidx], vals, semg).wait()
            prev_writes = [
                pltpu.async_copy(vals.at[pl.ds(c * BPW, BPW)],
                                 out.at[c0 + c, pl.ds(base, BPW)], semw)
                for c in range(FH)
            ]
    for w in prev_writes:
        w.wait()


_sc_gather3 = pl.kernel(
    _sc_gather3_body,
    out_type=[jax.ShapeDtypeStruct((D, B), jnp.float32)] * 3,
    mesh=_mesh,
    scratch_types=[
        pltpu.VMEM((3, BPW), jnp.int32),
        pltpu.VMEM((FH * BPW,), jnp.int32),
        pltpu.VMEM((FH * BPW,), jnp.float32),
        pltpu.SemaphoreType.DMA,
        pltpu.SemaphoreType.DMA,
    ],
    compiler_params=pltpu.CompilerParams(use_tc_tiling_on_sc=False,
                                         needs_layout_passes=False),
)

BS = 512


def _vq_body(x_ref, cb_ref, cbt_ref, c2_ref, q_ref, dsum_ref):
    x = x_ref[...]
    m = jnp.dot(2.0 * x, cb_ref[...], preferred_element_type=jnp.float32)
    x2 = jnp.sum(x * x, axis=1, keepdims=True)
    dist = (x2 - m) + c2_ref[...]
    rowmin = jnp.min(dist, axis=1, keepdims=True)
    eiota = lax.broadcasted_iota(jnp.int32, dist.shape, 1)
    idx = jnp.min(jnp.where(dist == rowmin, eiota, E), axis=1, keepdims=True)
    onehot = (eiota == idx).astype(jnp.float32)
    q_ref[...] = jnp.dot(onehot, cbt_ref[...],
                         precision=lax.Precision.HIGHEST,
                         preferred_element_type=jnp.float32)

    @pl.when(pl.program_id(0) == 0)
    def _():
        dsum_ref[0, 0] = 0.0

    dsum_ref[0, 0] += jnp.sum(rowmin)


_vq = pl.pallas_call(
    _vq_body,
    grid=(B // BS,),
    in_specs=[
        pl.BlockSpec((BS, D), lambda i: (i, 0)),
        pl.BlockSpec((D, E), lambda i: (0, 0)),
        pl.BlockSpec((E, D), lambda i: (0, 0)),
        pl.BlockSpec((1, E), lambda i: (0, 0)),
    ],
    out_specs=[
        pl.BlockSpec((BS, D), lambda i: (i, 0)),
        pl.BlockSpec((1, 1), lambda i: (0, 0), memory_space=pltpu.SMEM),
    ],
    out_shape=[
        jax.ShapeDtypeStruct((B, D), jnp.float32),
        jax.ShapeDtypeStruct((1, 1), jnp.float32),
    ],
)


def kernel(user_id, interacted_items, pos, neg, item_table, user_table, codebook):
    del interacted_items
    uid1 = user_id.astype(jnp.int32)
    pos1 = pos.astype(jnp.int32)
    neg1 = neg.astype(jnp.int32)
    userf = user_table.T.reshape(-1)
    itemf = item_table.T.reshape(-1)
    ueT, posT, negT = _sc_gather3(uid1, pos1, neg1, userf, itemf)
    user_embed = ueT.T
    c2 = jnp.sum(codebook ** 2, axis=0, keepdims=True)
    quant, dsum = _vq(user_embed, codebook, codebook.T, c2)
    diff = (dsum[0, 0] / (B * D)).astype(jnp.float32)
    return (quant, posT.T, negT.T, diff, user_embed)


# speedup vs baseline: 8.2343x; 8.2343x over previous
import jax
import jax.numpy as jnp
from jax import lax
from jax.experimental import pallas as pl
from jax.experimental.pallas import tpu as pltpu
from jax.experimental.pallas import tpu_sc as plsc

B = 16384
D = 64
E = 1024
NC = 2
NS = 16
NW = NC * NS
BPW = B // NW

_mesh = plsc.VectorSubcoreMesh(core_axis_name="c", subcore_axis_name="s")


def _wid():
    return lax.axis_index("s") * NC + lax.axis_index("c")


def _sc_gather3_body(uid_h, pos_h, neg_h, item_h, user_h,
                     ue_o, po_o, no_o,
                     uidx, pidx, nidx, urows, prows, nrows, sem):
    wid = _wid()
    base = wid * BPW
    sl = pl.ds(base, BPW)
    pltpu.sync_copy(uid_h.at[sl], uidx)
    pltpu.sync_copy(pos_h.at[sl], pidx)
    pltpu.sync_copy(neg_h.at[sl], nidx)
    cs = [pltpu.async_copy(user_h.at[uidx], urows, sem),
          pltpu.async_copy(item_h.at[pidx], prows, sem),
          pltpu.async_copy(item_h.at[nidx], nrows, sem)]
    for c in cs:
        c.wait()
    pltpu.sync_copy(urows, ue_o.at[sl])
    pltpu.sync_copy(prows, po_o.at[sl])
    pltpu.sync_copy(nrows, no_o.at[sl])


_sc_gather3 = pl.kernel(
    _sc_gather3_body,
    out_type=[jax.ShapeDtypeStruct((B, D), jnp.float32)] * 3,
    mesh=_mesh,
    scratch_types=[
        pltpu.VMEM((BPW,), jnp.int32),
        pltpu.VMEM((BPW,), jnp.int32),
        pltpu.VMEM((BPW,), jnp.int32),
        pltpu.VMEM((BPW, D), jnp.float32),
        pltpu.VMEM((BPW, D), jnp.float32),
        pltpu.VMEM((BPW, D), jnp.float32),
        pltpu.SemaphoreType.DMA,
    ],
    compiler_params=pltpu.CompilerParams(use_tc_tiling_on_sc=False),
)

BS = 512


def _vq_body(x_ref, cb_ref, cbt_ref, c2_ref, q_ref, dsum_ref):
    x = x_ref[...]
    m = jnp.dot(2.0 * x, cb_ref[...], preferred_element_type=jnp.float32)
    x2 = jnp.sum(x * x, axis=1, keepdims=True)
    dist = (x2 - m) + c2_ref[...]
    rowmin = jnp.min(dist, axis=1, keepdims=True)
    eiota = lax.broadcasted_iota(jnp.int32, dist.shape, 1)
    idx = jnp.min(jnp.where(dist == rowmin, eiota, E), axis=1, keepdims=True)
    onehot = (eiota == idx).astype(jnp.float32)
    q_ref[...] = jnp.dot(onehot, cbt_ref[...],
                         precision=lax.Precision.HIGHEST,
                         preferred_element_type=jnp.float32)

    @pl.when(pl.program_id(0) == 0)
    def _():
        dsum_ref[0, 0] = 0.0

    dsum_ref[0, 0] += jnp.sum(rowmin)


_vq = pl.pallas_call(
    _vq_body,
    grid=(B // BS,),
    in_specs=[
        pl.BlockSpec((BS, D), lambda i: (i, 0)),
        pl.BlockSpec((D, E), lambda i: (0, 0)),
        pl.BlockSpec((E, D), lambda i: (0, 0)),
        pl.BlockSpec((1, E), lambda i: (0, 0)),
    ],
    out_specs=[
        pl.BlockSpec((BS, D), lambda i: (i, 0)),
        pl.BlockSpec((1, 1), lambda i: (0, 0), memory_space=pltpu.SMEM),
    ],
    out_shape=[
        jax.ShapeDtypeStruct((B, D), jnp.float32),
        jax.ShapeDtypeStruct((1, 1), jnp.float32),
    ],
)


def kernel(user_id, interacted_items, pos, neg, item_table, user_table, codebook):
    del interacted_items
    uid1 = user_id.astype(jnp.int32)
    pos1 = pos.astype(jnp.int32)
    neg1 = neg.astype(jnp.int32)
    user_embed, pos_item, neg_item = _sc_gather3(
        uid1, pos1, neg1, item_table, user_table)
    c2 = jnp.sum(codebook ** 2, axis=0, keepdims=True)
    quant, dsum = _vq(user_embed, codebook, codebook.T, c2)
    diff = (dsum[0, 0] / (B * D)).astype(jnp.float32)
    return (quant, pos_item, neg_item, diff, user_embed)


# speedup vs baseline: 12.3563x; 1.5006x over previous
import jax
import jax.numpy as jnp
from jax import lax
from jax.experimental import pallas as pl
from jax.experimental.pallas import tpu as pltpu
from jax.experimental.pallas import tpu_sc as plsc

B = 16384
D = 64
E = 1024
NC = 2
NS = 16
NW = NC * NS
BPW = B // NW

_mesh = plsc.VectorSubcoreMesh(core_axis_name="c", subcore_axis_name="s")


def _wid():
    return lax.axis_index("s") * NC + lax.axis_index("c")


def _sc_gather3_body(uid_h, pos_h, neg_h, item_h, user_h,
                     ue_o, po_o, no_o,
                     idxv, rows, sem):
    wid = _wid()
    base = wid * BPW
    row0 = wid * (BPW // 128)
    for t, ih in enumerate((uid_h, pos_h, neg_h)):
        pltpu.sync_copy(ih.at[pl.ds(row0, BPW // 128)], idxv.at[t])
    for t, (tbl, out) in enumerate(
            ((user_h, ue_o), (item_h, po_o), (item_h, no_o))):

        def body(g, carry, tbl=tbl, t=t):
            q = lax.shift_right_logical(g, 3)
            k0 = lax.bitwise_and(g, 7) * 16
            v16 = idxv[t, q, pl.ds(k0, 16)]
            for l in range(16):
                r = v16[l]
                pltpu.async_copy(tbl.at[pl.ds(r, 1)],
                                 rows.at[pl.ds(g * 16 + l, 1)], sem)
            return carry

        lax.fori_loop(0, BPW // 16, body, 0)
        pltpu.make_async_copy(tbl.at[pl.ds(0, BPW)], rows, sem).wait()
        pltpu.sync_copy(rows, out.at[pl.ds(base, BPW)])


_sc_gather3 = pl.kernel(
    _sc_gather3_body,
    out_type=[jax.ShapeDtypeStruct((B, D), jnp.float32)] * 3,
    mesh=_mesh,
    scratch_types=[
        pltpu.VMEM((3, BPW // 128, 128), jnp.int32),
        pltpu.VMEM((BPW, D), jnp.float32),
        pltpu.SemaphoreType.DMA,
    ],
)

BS = 512


def _vq_body(x_ref, cb_ref, cbt_ref, c2_ref, q_ref, dsum_ref):
    x = x_ref[...]
    m = jnp.dot(2.0 * x, cb_ref[...], preferred_element_type=jnp.float32)
    x2 = jnp.sum(x * x, axis=1, keepdims=True)
    dist = (x2 - m) + c2_ref[...]
    rowmin = jnp.min(dist, axis=1, keepdims=True)
    eiota = lax.broadcasted_iota(jnp.int32, dist.shape, 1)
    idx = jnp.min(jnp.where(dist == rowmin, eiota, E), axis=1, keepdims=True)
    onehot = (eiota == idx).astype(jnp.float32)
    q_ref[...] = jnp.dot(onehot, cbt_ref[...],
                         precision=lax.Precision.HIGHEST,
                         preferred_element_type=jnp.float32)

    @pl.when(pl.program_id(0) == 0)
    def _():
        dsum_ref[0, 0] = 0.0

    dsum_ref[0, 0] += jnp.sum(rowmin)


_vq = pl.pallas_call(
    _vq_body,
    grid=(B // BS,),
    in_specs=[
        pl.BlockSpec((BS, D), lambda i: (i, 0)),
        pl.BlockSpec((D, E), lambda i: (0, 0)),
        pl.BlockSpec((E, D), lambda i: (0, 0)),
        pl.BlockSpec((1, E), lambda i: (0, 0)),
    ],
    out_specs=[
        pl.BlockSpec((BS, D), lambda i: (i, 0)),
        pl.BlockSpec((1, 1), lambda i: (0, 0), memory_space=pltpu.SMEM),
    ],
    out_shape=[
        jax.ShapeDtypeStruct((B, D), jnp.float32),
        jax.ShapeDtypeStruct((1, 1), jnp.float32),
    ],
)


def kernel(user_id, interacted_items, pos, neg, item_table, user_table, codebook):
    del interacted_items
    uid1 = user_id.astype(jnp.int32).reshape(B // 128, 128)
    pos1 = pos.astype(jnp.int32).reshape(B // 128, 128)
    neg1 = neg.astype(jnp.int32).reshape(B // 128, 128)
    user_embed, pos_item, neg_item = _sc_gather3(
        uid1, pos1, neg1, item_table, user_table)
    c2 = jnp.sum(codebook ** 2, axis=0, keepdims=True)
    quant, dsum = _vq(user_embed, codebook, codebook.T, c2)
    diff = (dsum[0, 0] / (B * D)).astype(jnp.float32)
    return (quant, pos_item, neg_item, diff, user_embed)
